# Initial kernel scaffold; baseline (speedup 1.0000x reference)
import jax
import jax.numpy as jnp
from jax.experimental import pallas as pl


def kernel(tokens, tables, weights):
    raise NotImplementedError("write your pallas kernel here")


# speedup vs baseline: 5.6558x; 5.6558x over previous
import functools

import jax
import jax.numpy as jnp
from jax import lax
from jax.experimental import pallas as pl
from jax.experimental.pallas import tpu as pltpu
from jax.experimental.pallas import tpu_sc as plsc

NQ = 8
V = 100000
D = 32
B = 4096
T = 50
BT = B * T
NW = 32
PER_W = BT // NW
CH = 128
NCH = PER_W // CH


@functools.partial(
    pl.kernel,
    mesh=plsc.VectorSubcoreMesh(core_axis_name="c", subcore_axis_name="s"),
    out_type=jax.ShapeDtypeStruct((BT, D), jnp.float32),
    compiler_params=pltpu.CompilerParams(use_tc_tiling_on_sc=False),
    scratch_types=[
        pltpu.VMEM((NQ, 16), jnp.float32),
        pltpu.VMEM((CH,), jnp.int32),
        pltpu.VMEM((CH, D), jnp.float32),
        pltpu.VMEM((CH, D), jnp.float32),
        pltpu.SemaphoreType.DMA,
    ],
)
def _wce(tok_hbm, tab_hbm, w_hbm, out_hbm, w_v, idx_v, rows_v, acc_v, sem):
    wid = lax.axis_index("s") * 2 + lax.axis_index("c")
    pltpu.sync_copy(w_hbm, w_v)

    def chunk_body(c, carry):
        base = pl.multiple_of(wid * PER_W + c * CH, CH)
        for i in range(NQ):
            pltpu.sync_copy(tok_hbm.at[i, pl.ds(base, CH)], idx_v)
            off = jnp.full((16,), i * V, jnp.int32)

            def add_off(k, carry2):
                o = pl.multiple_of(k * 16, 16)
                idx_v[pl.ds(o, 16)] = idx_v[pl.ds(o, 16)] + off
                return carry2

            lax.fori_loop(0, CH // 16, add_off, 0)
            pltpu.async_copy(tab_hbm.at[idx_v], rows_v, sem).wait()
            w_i = w_v[i, :]

            def accum(r8, carry2):
                for j in range(8):
                    r = r8 * 8 + j
                    for cc in range(D // 16):
                        v = rows_v[r, pl.ds(cc * 16, 16)] * w_i
                        if i == 0:
                            acc_v[r, pl.ds(cc * 16, 16)] = v
                        else:
                            plsc.addupdate(acc_v.at[r, pl.ds(cc * 16, 16)], v)
                return carry2

            lax.fori_loop(0, CH // 8, accum, 0)
        pltpu.sync_copy(acc_v, out_hbm.at[pl.ds(base, CH)])
        return carry

    lax.fori_loop(0, NCH, chunk_body, 0)


def kernel(tokens, tables, weights):
    tok_t = tokens.reshape(B, NQ, T).transpose(1, 0, 2).reshape(NQ, BT)
    tab_f = tables.reshape(NQ * V, D)
    w_rep = jnp.broadcast_to(weights[:, None], (NQ, 16))
    out = _wce(tok_t, tab_f, w_rep)
    return out.reshape(B, T, D)


# speedup vs baseline: 8.3334x; 1.4734x over previous
import functools

import jax
import jax.numpy as jnp
from jax import lax
from jax.experimental import pallas as pl
from jax.experimental.pallas import tpu as pltpu
from jax.experimental.pallas import tpu_sc as plsc

NQ = 8
V = 100000
D = 32
B = 4096
T = 50
BT = B * T
NW = 32
CH = 128
NCH_W = BT // (NW * CH)


@functools.partial(
    pl.kernel,
    mesh=plsc.VectorSubcoreMesh(core_axis_name="c", subcore_axis_name="s"),
    out_type=jax.ShapeDtypeStruct((BT, D), jnp.float32),
    compiler_params=pltpu.CompilerParams(use_tc_tiling_on_sc=False),
    scratch_types=[
        pltpu.VMEM((NQ, 16), jnp.float32),
        pltpu.VMEM((2, NQ, CH), jnp.int32),
        pltpu.VMEM((2, NQ, CH, D), jnp.float32),
        pltpu.VMEM((2, CH, D), jnp.float32),
        pltpu.SemaphoreType.DMA,
        pltpu.SemaphoreType.DMA,
        pltpu.SemaphoreType.DMA,
        pltpu.SemaphoreType.DMA,
    ],
)
def _wce(tok_hbm, tab_hbm, w_hbm, out_hbm,
         w_v, idx_v, rows_v, acc_v, gsem0, gsem1, osem0, osem1):
    wid = lax.axis_index("s") * 2 + lax.axis_index("c")
    pltpu.sync_copy(w_hbm, w_v)
    c0 = wid * NCH_W
    gsems = (gsem0, gsem1)
    osems = (osem0, osem1)

    def fire(cg, b):
        pltpu.sync_copy(tok_hbm.at[cg], idx_v.at[b])
        for i in range(NQ):
            pltpu.async_copy(
                tab_hbm.at[i].at[idx_v.at[b, i]], rows_v.at[b, i], gsems[b])

    def drain_gathers(b):
        for i in range(NQ):
            pltpu.make_async_copy(
                tab_hbm.at[i].at[idx_v.at[b, i]], rows_v.at[b, i],
                gsems[b]).wait()

    def accum_store(cg, b, first):
        @pl.when(jnp.logical_not(first))
        def _():
            pltpu.make_async_copy(
                acc_v.at[b], out_hbm.at[pl.ds(0, CH)], osems[b]).wait()

        def rowbody(r2, carry):
            for rr in range(2):
                r = r2 * 2 + rr
                for c16 in range(D // 16):
                    v = rows_v[b, 0, r, pl.ds(c16 * 16, 16)] * w_v[0, :]
                    for i in range(1, NQ):
                        v = v + rows_v[b, i, r, pl.ds(c16 * 16, 16)] * w_v[i, :]
                    acc_v[b, r, pl.ds(c16 * 16, 16)] = v
            return carry

        lax.fori_loop(0, CH // 2, rowbody, 0)
        base = pl.multiple_of(cg * CH, CH)
        pltpu.async_copy(acc_v.at[b], out_hbm.at[pl.ds(base, CH)], osems[b])

    fire(c0, 0)

    def pair(p, carry):
        cga = c0 + 2 * p
        fire(cga + 1, 1)
        drain_gathers(0)
        accum_store(cga, 0, p == 0)

        @pl.when(p < NCH_W // 2 - 1)
        def _():
            fire(cga + 2, 0)

        drain_gathers(1)
        accum_store(cga + 1, 1, p == 0)
        return carry

    lax.fori_loop(0, NCH_W // 2, pair, 0)
    pltpu.make_async_copy(acc_v.at[0], out_hbm.at[pl.ds(0, CH)], osem0).wait()
    pltpu.make_async_copy(acc_v.at[1], out_hbm.at[pl.ds(0, CH)], osem1).wait()


def kernel(tokens, tables, weights):
    tok_t = tokens.reshape(B, NQ, T).transpose(1, 0, 2)
    tok_r = tok_t.reshape(NQ, BT // CH, CH).transpose(1, 0, 2)
    w_rep = jnp.broadcast_to(weights[:, None], (NQ, 16))
    out = _wce(tok_r, tables, w_rep)
    return out.reshape(B, T, D)


# speedup vs baseline: 10.2513x; 1.2301x over previous
import functools

import jax
import jax.numpy as jnp
from jax import lax
from jax.experimental import pallas as pl
from jax.experimental.pallas import tpu as pltpu
from jax.experimental.pallas import tpu_sc as plsc

NQ = 8
V = 100000
D = 32
B = 4096
T = 50
NW = 32
NB_W = B // NW
G = 4
GK = G * NQ * T
NCH_W = NB_W // G
SEGS = [(s, 128) for s in range(0, 1536, 128)] + [(1536, 64)]


@functools.partial(
    pl.kernel,
    mesh=plsc.VectorSubcoreMesh(core_axis_name="c", subcore_axis_name="s"),
    out_type=jax.ShapeDtypeStruct((B, T, D), jnp.float32),
    compiler_params=pltpu.CompilerParams(use_tc_tiling_on_sc=False),
    scratch_types=[
        pltpu.VMEM((NQ, 16), jnp.float32),
        pltpu.VMEM((2, G, NQ * T), jnp.int32),
        pltpu.VMEM((2, GK), jnp.int32),
        pltpu.VMEM((2, GK, D), jnp.float32),
        pltpu.VMEM((2, G, T, D), jnp.float32),
        pltpu.SemaphoreType.DMA,
        pltpu.SemaphoreType.DMA,
        pltpu.SemaphoreType.DMA,
        pltpu.SemaphoreType.DMA,
    ],
)
def _wce(tok_hbm, tab_hbm, w_hbm, out_hbm,
         w_v, tokbuf_v, idx_v, rows_v, acc_v,
         gsem0, gsem1, osem0, osem1):
    wid = lax.axis_index("s") * 2 + lax.axis_index("c")
    pltpu.sync_copy(w_hbm, w_v)
    gsems = (gsem0, gsem1)
    osems = (osem0, osem1)

    l16 = jnp.arange(16, dtype=jnp.int32)

    def offs(qq):
        k0 = qq * 16
        c_lo = k0 // T
        l_b = T - (k0 % T)
        lo = jnp.full((16,), c_lo * V, jnp.int32)
        if l_b >= 16:
            return lo
        hi = jnp.full((16,), (c_lo + 1) * V, jnp.int32)
        return jnp.where(l16 >= l_b, hi, lo)

    def fire(b, brow):
        pltpu.sync_copy(tok_hbm.at[pl.ds(brow, G)], tokbuf_v.at[b])
        for q in range(GK // 16):
            idx_v[b, pl.ds(q * 16, 16)] = (
                tokbuf_v[b, q // 25, pl.ds((q % 25) * 16, 16)]
                + offs(q % 25))
        for s, n in SEGS:
            pltpu.async_copy(
                tab_hbm.at[idx_v.at[b, pl.ds(s, n)]],
                rows_v.at[b, pl.ds(s, n)], gsems[b])

    def drain(b):
        for s, n in SEGS:
            pltpu.make_async_copy(
                tab_hbm.at[idx_v.at[b, pl.ds(s, n)]],
                rows_v.at[b, pl.ds(s, n)], gsems[b]).wait()

    def accum_store(b, brow, first):
        @pl.when(jnp.logical_not(first))
        def _():
            pltpu.make_async_copy(
                acc_v.at[b], out_hbm.at[pl.ds(0, G)], osems[b]).wait()

        def rowbody(t, carry):
            for g in range(G):
                for c16 in range(D // 16):
                    cs = c16 * 16
                    kk = g * (NQ * T) + t
                    v = rows_v[b, kk, pl.ds(cs, 16)] * w_v[0, :]
                    for i in range(1, NQ):
                        v = v + rows_v[b, kk + i * T, pl.ds(cs, 16)] * w_v[i, :]
                    acc_v[b, g, t, pl.ds(cs, 16)] = v
            return carry

        lax.fori_loop(0, T, rowbody, 0)
        pltpu.async_copy(acc_v.at[b], out_hbm.at[pl.ds(brow, G)], osems[b])

    c0 = wid * NB_W
    fire(0, c0)

    def pair(p, carry):
        browa = c0 + 2 * p * G
        fire(1, browa + G)
        drain(0)
        accum_store(0, browa, p == 0)

        @pl.when(p < NCH_W // 2 - 1)
        def _():
            fire(0, browa + 2 * G)

        drain(1)
        accum_store(1, browa + G, p == 0)
        return carry

    lax.fori_loop(0, NCH_W // 2, pair, 0)
    pltpu.make_async_copy(acc_v.at[0], out_hbm.at[pl.ds(0, G)], osem0).wait()
    pltpu.make_async_copy(acc_v.at[1], out_hbm.at[pl.ds(0, G)], osem1).wait()


def kernel(tokens, tables, weights):
    tab_f = tables.reshape(NQ * V, D)
    w_rep = jnp.broadcast_to(weights[:, None], (NQ, 16))
    return _wce(tokens, tab_f, w_rep)


# speedup vs baseline: 11.4271x; 1.1147x over previous
import functools

import jax
import jax.numpy as jnp
from jax import lax
from jax.experimental import pallas as pl
from jax.experimental.pallas import tpu as pltpu
from jax.experimental.pallas import tpu_sc as plsc

NQ = 8
V = 100000
D = 32
B = 4096
T = 50
NW = 32
W = B // NW


@functools.partial(
    pl.kernel,
    mesh=plsc.VectorSubcoreMesh(core_axis_name="c", subcore_axis_name="s"),
    out_type=jax.ShapeDtypeStruct((T, D, B), jnp.float32),
    compiler_params=pltpu.CompilerParams(use_tc_tiling_on_sc=False,
                                         needs_layout_passes=False),
    scratch_types=[
        pltpu.VMEM((NQ, 16), jnp.float32),
        pltpu.VMEM((NQ * T, W), jnp.int32),
        pltpu.VMEM((2, NQ, W, D), jnp.float32),
        pltpu.VMEM((2, D, W), jnp.float32),
        pltpu.SemaphoreType.DMA,
        pltpu.SemaphoreType.DMA,
        pltpu.SemaphoreType.DMA,
        pltpu.SemaphoreType.DMA,
    ],
)
def _wce(tok_hbm, tab_hbm, w_hbm, out_hbm,
         w_v, toks_v, rows_v, acc_v, gsem0, gsem1, osem0, osem1):
    wid = lax.axis_index("s") * 2 + lax.axis_index("c")
    b0 = pl.multiple_of(wid * W, W)
    pltpu.sync_copy(w_hbm, w_v)
    pltpu.sync_copy(tok_hbm.at[:, pl.ds(b0, W)], toks_v)
    gsems = (gsem0, gsem1)
    osems = (osem0, osem1)
    ci = jnp.arange(16, dtype=jnp.int32)

    def fire(b, t0):
        for i in range(NQ):
            pltpu.async_copy(
                tab_hbm.at[i].at[toks_v.at[i * T + t0]],
                rows_v.at[b, i], gsems[b])

    def drain(b, t0):
        for i in range(NQ):
            pltpu.make_async_copy(
                tab_hbm.at[i].at[toks_v.at[i * T + t0]],
                rows_v.at[b, i], gsems[b]).wait()

    def accum_store(b, t0, first):
        @pl.when(jnp.logical_not(first))
        def _():
            pltpu.make_async_copy(
                acc_v.at[b], out_hbm.at[0, :, pl.ds(0, W)], osems[b]).wait()

        def rowbody(j, carry):
            lo = rows_v[b, 0, j, pl.ds(0, 16)] * w_v[0, :]
            hi = rows_v[b, 0, j, pl.ds(16, 16)] * w_v[0, :]
            for i in range(1, NQ):
                lo = lo + rows_v[b, i, j, pl.ds(0, 16)] * w_v[i, :]
                hi = hi + rows_v[b, i, j, pl.ds(16, 16)] * w_v[i, :]
            js = jnp.full((16,), 0, jnp.int32) + j
            plsc.store_scatter(acc_v.at[b], [ci, js], lo)
            plsc.store_scatter(acc_v.at[b], [ci + 16, js], hi)
            return carry

        lax.fori_loop(0, W, rowbody, 0)
        pltpu.async_copy(acc_v.at[b], out_hbm.at[t0, :, pl.ds(b0, W)],
                         osems[b])

    fire(0, 0)

    def pair(p, carry):
        t0a = 2 * p
        fire(1, t0a + 1)
        drain(0, t0a)
        accum_store(0, t0a, p == 0)

        @pl.when(p < T // 2 - 1)
        def _():
            fire(0, t0a + 2)

        drain(1, t0a + 1)
        accum_store(1, t0a + 1, p == 0)
        return carry

    lax.fori_loop(0, T // 2, pair, 0)
    pltpu.make_async_copy(
        acc_v.at[0], out_hbm.at[0, :, pl.ds(0, W)], osem0).wait()
    pltpu.make_async_copy(
        acc_v.at[1], out_hbm.at[0, :, pl.ds(0, W)], osem1).wait()


def kernel(tokens, tables, weights):
    tok_nat = jnp.transpose(tokens)
    w_rep = jnp.broadcast_to(weights[:, None], (NQ, 16))
    out_p = _wce(tok_nat, tables, w_rep)
    return jnp.transpose(out_p, (2, 0, 1))
